# baseline (speedup 1.0000x reference)
import functools

import jax
import jax.numpy as jnp
from jax import lax
from jax.experimental import pallas as pl
from jax.experimental.pallas import tpu as pltpu
from jax.experimental.pallas import tpu_sc as plsc

_L = 16


def _tc_kernel(p1_ref, p2_ref, row_acc, col_acc, *, ti, tj, ni, nj):
    i = pl.program_id(0)
    j = pl.program_id(1)

    p1 = p1_ref[...]
    p2 = p2_ref[...]

    acc = jnp.zeros((ti, tj), jnp.float32)
    for d in range(3):
        diff = p1[:, d][:, None] - p2[d, :][None, :]
        acc = acc + diff * diff

    row_m = jnp.min(acc, axis=1)[:, None]
    col_m = jnp.min(acc, axis=0)[None, :]

    @pl.when((i == 0) & (j == 0))
    def _():
        row_acc[...] = jnp.full(row_acc.shape, jnp.inf, jnp.float32)
        col_acc[...] = jnp.full(col_acc.shape, jnp.inf, jnp.float32)

    row_acc[pl.ds(i * ti, ti), :] = jnp.minimum(
        row_acc[pl.ds(i * ti, ti), :], row_m)
    col_acc[:, pl.ds(j * tj, tj)] = jnp.minimum(
        col_acc[:, pl.ds(j * tj, tj)], col_m)


def _sc_kernel(q_hbm, x2_hbm, y2_hbm, z2_hbm, rowm_hbm, colp_hbm,
               q_v, x2_v, y2_v, z2_v, colp_v, rowm_v, *,
               rows_per_w, msc, nc):
    wid = lax.axis_index("s") * nc + lax.axis_index("c")
    base = wid * rows_per_w

    pltpu.sync_copy(q_hbm.at[pl.ds(base, rows_per_w), :], q_v)
    pltpu.sync_copy(x2_hbm, x2_v)
    pltpu.sync_copy(y2_hbm, y2_v)
    pltpu.sync_copy(z2_hbm, z2_v)

    nchunk = msc // _L
    inf16 = jnp.full((_L,), jnp.inf, jnp.float32)

    def init_body(c, carry):
        colp_v[pl.ds(c * _L, _L)] = inf16
        return carry
    lax.fori_loop(0, nchunk, init_body, 0)

    def query_body(q, carry):
        xq = q_v[q, 0:_L]
        yq = q_v[q, _L:2 * _L]
        zq = q_v[q, 2 * _L:3 * _L]

        def chunk_body(c, best):
            s = c * _L
            dx = xq - x2_v[pl.ds(s, _L)]
            dy = yq - y2_v[pl.ds(s, _L)]
            dz = zq - z2_v[pl.ds(s, _L)]
            d2 = dx * dx + dy * dy + dz * dz
            colp_v[pl.ds(s, _L)] = jnp.minimum(colp_v[pl.ds(s, _L)], d2)
            return jnp.minimum(best, d2)

        best = lax.fori_loop(0, nchunk, chunk_body, inf16, unroll=4)
        rowm_v[q, :] = best
        return carry

    lax.fori_loop(0, rows_per_w, query_body, 0)

    pltpu.sync_copy(rowm_v, rowm_hbm.at[pl.ds(base, rows_per_w), :])
    pltpu.sync_copy(colp_v, colp_hbm.at[wid])


def _combine_kernel(rowtc_ref, coltc_ref, rowsc_ref, colsc_ref, out_ref, *,
                    n1, n2, npad1, mtc, msc):
    rm = jnp.minimum(rowtc_ref[...],
                     jnp.min(rowsc_ref[...], axis=1)[:, None])
    rvalid = jax.lax.broadcasted_iota(jnp.int32, (npad1, 1), 0) < n1
    s1 = jnp.sum(jnp.where(rvalid, jnp.sqrt(rm), 0.0))
    s2a = jnp.sum(jnp.sqrt(coltc_ref[...]))
    cm = jnp.min(colsc_ref[...], axis=0)[None, :]
    cvalid = jax.lax.broadcasted_iota(jnp.int32, (1, msc), 1) < (n2 - mtc)
    s2b = jnp.sum(jnp.where(cvalid, jnp.sqrt(cm), 0.0))
    out_ref[...] = (s1 + s2a + s2b)[None, None]


def kernel(points1, points2):
    n1 = points1.shape[0]
    n2 = points2.shape[0]
    p1 = points1.astype(jnp.float32)
    p2 = points2.astype(jnp.float32)

    info = plsc.get_sparse_core_info()
    nc, ns = info.num_cores, info.num_subcores
    nw = nc * ns

    ti = 2560
    tj = 2048
    npad1 = ((n1 + (nw * _L) - 1) // (nw * _L)) * (nw * _L)
    rows_per_w = npad1 // nw
    ni = npad1 // ti
    assert npad1 % ti == 0

    mtc = (n2 // tj) * tj
    if mtc >= n2:
        mtc -= tj
    nj = mtc // tj
    nsc = n2 - mtc
    msc = ((nsc + _L - 1) // _L) * _L

    p1p = jnp.zeros((npad1, 8), jnp.float32)
    p1p = p1p.at[:n1, :3].set(p1)
    p1p = p1p.at[n1:, :3].set(jnp.inf)
    p2p = jnp.zeros((8, mtc), jnp.float32)
    p2p = p2p.at[:3, :].set(p2[:mtc, :].T)

    q = jnp.full((npad1, 3), jnp.inf, jnp.float32).at[:n1, :].set(p1)
    qrep = jnp.repeat(q, _L, axis=1)

    def sc_coord(col):
        return jnp.full((msc,), jnp.inf,
                        jnp.float32).at[:nsc].set(p2[mtc:, col])

    x2 = sc_coord(0)
    y2 = sc_coord(1)
    z2 = sc_coord(2)

    row_tc, col_tc = pl.pallas_call(
        functools.partial(_tc_kernel, ti=ti, tj=tj, ni=ni, nj=nj),
        grid=(ni, nj),
        in_specs=[
            pl.BlockSpec((ti, 8), lambda i, j: (i, 0)),
            pl.BlockSpec((8, tj), lambda i, j: (0, j)),
        ],
        out_specs=[
            pl.BlockSpec((npad1, 1), lambda i, j: (0, 0)),
            pl.BlockSpec((1, mtc), lambda i, j: (0, 0)),
        ],
        out_shape=[
            jax.ShapeDtypeStruct((npad1, 1), jnp.float32),
            jax.ShapeDtypeStruct((1, mtc), jnp.float32),
        ],
        compiler_params=pltpu.CompilerParams(
            dimension_semantics=("arbitrary", "arbitrary"),
        ),
    )(p1p, p2p)

    mesh = plsc.VectorSubcoreMesh(core_axis_name="c", subcore_axis_name="s")
    sc = functools.partial(
        pl.kernel,
        mesh=mesh,
        out_type=[
            jax.ShapeDtypeStruct((npad1, _L), jnp.float32),
            jax.ShapeDtypeStruct((nw, msc), jnp.float32),
        ],
        scratch_types=[
            pltpu.VMEM((rows_per_w, 3 * _L), jnp.float32),
            pltpu.VMEM((msc,), jnp.float32),
            pltpu.VMEM((msc,), jnp.float32),
            pltpu.VMEM((msc,), jnp.float32),
            pltpu.VMEM((msc,), jnp.float32),
            pltpu.VMEM((rows_per_w, _L), jnp.float32),
        ],
    )(functools.partial(_sc_kernel, rows_per_w=rows_per_w, msc=msc, nc=nc))
    row_sc, col_sc = sc(qrep, x2, y2, z2)

    out = pl.pallas_call(
        functools.partial(_combine_kernel, n1=n1, n2=n2,
                          npad1=npad1, mtc=mtc, msc=msc),
        in_specs=[
            pl.BlockSpec((npad1, 1), lambda: (0, 0)),
            pl.BlockSpec((1, mtc), lambda: (0, 0)),
            pl.BlockSpec((npad1, _L), lambda: (0, 0)),
            pl.BlockSpec((nw, msc), lambda: (0, 0)),
        ],
        out_specs=pl.BlockSpec((1, 1), lambda: (0, 0)),
        out_shape=jax.ShapeDtypeStruct((1, 1), jnp.float32),
    )(row_tc, col_tc, row_sc, col_sc)
    return out[0, 0]


# speedup vs baseline: 1.9974x; 1.9974x over previous
import functools

import jax
import jax.numpy as jnp
from jax import lax
from jax.experimental import pallas as pl
from jax.experimental.pallas import tpu as pltpu
from jax.experimental.pallas import tpu_sc as plsc

_L = 16


def _tc_kernel(p1_ref, p2_ref, row_acc, col_acc, *, ti, tj, ni, nj):
    i = pl.program_id(0)
    j = pl.program_id(1)

    p1 = p1_ref[...]
    p2 = p2_ref[...]

    acc = jnp.zeros((ti, tj), jnp.float32)
    for d in range(3):
        diff = p1[:, d][:, None] - p2[d, :][None, :]
        acc = acc + diff * diff

    row_m = jnp.min(acc, axis=1)[:, None]
    col_m = jnp.min(acc, axis=0)[None, :]

    @pl.when((i == 0) & (j == 0))
    def _():
        row_acc[...] = jnp.full(row_acc.shape, jnp.inf, jnp.float32)
        col_acc[...] = jnp.full(col_acc.shape, jnp.inf, jnp.float32)

    row_acc[pl.ds(i * ti, ti), :] = jnp.minimum(
        row_acc[pl.ds(i * ti, ti), :], row_m)
    col_acc[:, pl.ds(j * tj, tj)] = jnp.minimum(
        col_acc[:, pl.ds(j * tj, tj)], col_m)


def _sc_kernel(q_hbm, x2_hbm, y2_hbm, z2_hbm, rowm_hbm, colp_hbm,
               q_v, x2_v, y2_v, z2_v, colp_v, rowm_v, *,
               rows_per_w, msc, nc):
    wid = lax.axis_index("s") * nc + lax.axis_index("c")
    base = wid * rows_per_w

    pltpu.sync_copy(q_hbm.at[pl.ds(base, rows_per_w), :], q_v)
    pltpu.sync_copy(x2_hbm, x2_v)
    pltpu.sync_copy(y2_hbm, y2_v)
    pltpu.sync_copy(z2_hbm, z2_v)

    nchunk = msc // _L
    inf16 = jnp.full((_L,), jnp.inf, jnp.float32)

    def init_body(c, carry):
        colp_v[pl.ds(c * _L, _L)] = inf16
        return carry
    lax.fori_loop(0, nchunk, init_body, 0)

    def query_body(q, carry):
        xq = q_v[q, 0:_L]
        yq = q_v[q, _L:2 * _L]
        zq = q_v[q, 2 * _L:3 * _L]

        def chunk_body(c, best):
            s = c * _L
            dx = xq - x2_v[pl.ds(s, _L)]
            dy = yq - y2_v[pl.ds(s, _L)]
            dz = zq - z2_v[pl.ds(s, _L)]
            d2 = dx * dx + dy * dy + dz * dz
            colp_v[pl.ds(s, _L)] = jnp.minimum(colp_v[pl.ds(s, _L)], d2)
            return jnp.minimum(best, d2)

        best = lax.fori_loop(0, nchunk, chunk_body, inf16)
        rowm_v[q, :] = best
        return carry

    lax.fori_loop(0, rows_per_w, query_body, 0)

    pltpu.sync_copy(rowm_v, rowm_hbm.at[pl.ds(base, rows_per_w), :])
    pltpu.sync_copy(colp_v, colp_hbm.at[wid])


def _combine_kernel(rowtc_ref, coltc_ref, rowsc_ref, colsc_ref, out_ref, *,
                    n1, n2, npad1, mtc, msc):
    rm = jnp.minimum(rowtc_ref[...],
                     jnp.min(rowsc_ref[...], axis=1)[:, None])
    rvalid = jax.lax.broadcasted_iota(jnp.int32, (npad1, 1), 0) < n1
    s1 = jnp.sum(jnp.where(rvalid, jnp.sqrt(rm), 0.0))
    s2a = jnp.sum(jnp.sqrt(coltc_ref[...]))
    cm = jnp.min(colsc_ref[...], axis=0)[None, :]
    cvalid = jax.lax.broadcasted_iota(jnp.int32, (1, msc), 1) < (n2 - mtc)
    s2b = jnp.sum(jnp.where(cvalid, jnp.sqrt(cm), 0.0))
    out_ref[...] = (s1 + s2a + s2b)[None, None]


def kernel(points1, points2):
    n1 = points1.shape[0]
    n2 = points2.shape[0]
    p1 = points1.astype(jnp.float32)
    p2 = points2.astype(jnp.float32)

    info = plsc.get_sparse_core_info()
    nc, ns = info.num_cores, info.num_subcores
    nw = nc * ns

    ti = 2560
    tj = 2048
    npad1 = ((n1 + (nw * _L) - 1) // (nw * _L)) * (nw * _L)
    rows_per_w = npad1 // nw
    ni = npad1 // ti
    assert npad1 % ti == 0

    mtc = (n2 // tj) * tj
    if mtc >= n2:
        mtc -= tj
    nj = mtc // tj
    nsc = n2 - mtc
    msc = ((nsc + _L - 1) // _L) * _L

    p1p = jnp.zeros((npad1, 8), jnp.float32)
    p1p = p1p.at[:n1, :3].set(p1)
    p1p = p1p.at[n1:, :3].set(jnp.inf)
    p2p = jnp.zeros((8, mtc), jnp.float32)
    p2p = p2p.at[:3, :].set(p2[:mtc, :].T)

    q = jnp.full((npad1, 3), jnp.inf, jnp.float32).at[:n1, :].set(p1)
    qrep = jnp.repeat(q, _L, axis=1)

    def sc_coord(col):
        return jnp.full((msc,), jnp.inf,
                        jnp.float32).at[:nsc].set(p2[mtc:, col])

    x2 = sc_coord(0)
    y2 = sc_coord(1)
    z2 = sc_coord(2)

    row_tc, col_tc = pl.pallas_call(
        functools.partial(_tc_kernel, ti=ti, tj=tj, ni=ni, nj=nj),
        grid=(ni, nj),
        in_specs=[
            pl.BlockSpec((ti, 8), lambda i, j: (i, 0)),
            pl.BlockSpec((8, tj), lambda i, j: (0, j)),
        ],
        out_specs=[
            pl.BlockSpec((npad1, 1), lambda i, j: (0, 0)),
            pl.BlockSpec((1, mtc), lambda i, j: (0, 0)),
        ],
        out_shape=[
            jax.ShapeDtypeStruct((npad1, 1), jnp.float32),
            jax.ShapeDtypeStruct((1, mtc), jnp.float32),
        ],
        compiler_params=pltpu.CompilerParams(
            dimension_semantics=("arbitrary", "arbitrary"),
        ),
    )(p1p, p2p)

    mesh = plsc.VectorSubcoreMesh(core_axis_name="c", subcore_axis_name="s")
    sc = functools.partial(
        pl.kernel,
        mesh=mesh,
        out_type=[
            jax.ShapeDtypeStruct((npad1, _L), jnp.float32),
            jax.ShapeDtypeStruct((nw, msc), jnp.float32),
        ],
        scratch_types=[
            pltpu.VMEM((rows_per_w, 3 * _L), jnp.float32),
            pltpu.VMEM((msc,), jnp.float32),
            pltpu.VMEM((msc,), jnp.float32),
            pltpu.VMEM((msc,), jnp.float32),
            pltpu.VMEM((msc,), jnp.float32),
            pltpu.VMEM((rows_per_w, _L), jnp.float32),
        ],
    )(functools.partial(_sc_kernel, rows_per_w=rows_per_w, msc=msc, nc=nc))
    row_sc, col_sc = sc(qrep, x2, y2, z2)

    out = pl.pallas_call(
        functools.partial(_combine_kernel, n1=n1, n2=n2,
                          npad1=npad1, mtc=mtc, msc=msc),
        in_specs=[
            pl.BlockSpec((npad1, 1), lambda: (0, 0)),
            pl.BlockSpec((1, mtc), lambda: (0, 0)),
            pl.BlockSpec((npad1, _L), lambda: (0, 0)),
            pl.BlockSpec((nw, msc), lambda: (0, 0)),
        ],
        out_specs=pl.BlockSpec((1, 1), lambda: (0, 0)),
        out_shape=jax.ShapeDtypeStruct((1, 1), jnp.float32),
    )(row_tc, col_tc, row_sc, col_sc)
    return out[0, 0]
